# baseline (speedup 1.0000x reference)
import functools

import jax
import jax.numpy as jnp
from jax import lax
from jax.experimental import pallas as pl
from jax.experimental.pallas import tpu as pltpu
from jax.experimental.pallas import tpu_sc as plsc

B = 1024
L = 20
D = 128
LAYER_SIZES = [512, 256, 128]
SCALAR_FEATS = ['pl_collaborative', 'pl_pid', 'duration_ms_seed', 'n_songs',
                'n_artists', 'n_albums']
SEQ_FEATS = ['artist_name', 'track_uri', 'track_name', 'duration_ms_songs',
             'album_name', 'artist_pop', 'artists_followers', 'track_pop',
             'artist_genres']

NC = 2
NS = 16
NW = NC * NS
NBIG = 10
ROWS_BIG = L * B
KCH = ROWS_BIG // NW // 128
NSC = 6


RING = 4
NTASK = NBIG * KCH


def _sc_gather_body(*refs):
    tabs = refs[0:NBIG]
    idx_all = refs[NBIG]
    stabs = refs[NBIG + 1:NBIG + 1 + NSC]
    sidxs = refs[NBIG + 1 + NSC:NBIG + 1 + 2 * NSC]
    outs = refs[NBIG + 1 + 2 * NSC:2 * NBIG + 1 + 2 * NSC]
    souts = refs[2 * NBIG + 1 + 2 * NSC:2 * NBIG + 1 + 3 * NSC]
    idx_v, rows_v, idx_s, rows_s, gsem, ssem, s2sem = \
        refs[2 * NBIG + 1 + 3 * NSC:]

    w = lax.axis_index("s") * NC + lax.axis_index("c")

    pltpu.sync_copy(idx_all.at[w], idx_v)

    def gcopy(i):
        f = i // KCH
        return pltpu.make_async_copy(tabs[f].at[idx_v.at[i]],
                                     rows_v.at[i % RING], gsem)

    def scopy(i):
        f, k = divmod(i, KCH)
        return pltpu.make_async_copy(rows_v.at[i % RING],
                                     outs[f].at[w * KCH + k], ssem)

    for i in range(RING):
        gcopy(i).start()
    for i in range(NTASK):
        gcopy(i).wait()
        scopy(i).start()
        scopy(i).wait()
        if i + RING < NTASK:
            gcopy(i + RING).start()

    @pl.when(w < 8)
    def _():
        def sg(f, b):
            pltpu.sync_copy(sidxs[f].at[w], idx_s.at[pl.ds(b, 1)])
            pltpu.make_async_copy(stabs[f].at[idx_s.at[b]],
                                  rows_s.at[b], s2sem).start()

        def sg_wait(f, b):
            pltpu.make_async_copy(stabs[f].at[idx_s.at[b]],
                                  rows_s.at[b], s2sem).wait()

        sg(0, 0)
        for f in range(NSC):
            b = f % 2
            sg_wait(f, b)
            if f + 1 < NSC:
                sg(f + 1, 1 - b)
            pltpu.sync_copy(rows_s.at[b], souts[f].at[pl.ds(w * 128, 128)])


def _sc_gather(tabs, idx_all, stabs, sidxs):
    out_type = ([jax.ShapeDtypeStruct((ROWS_BIG // 128, 128, D), jnp.float32)
                 for _ in range(NBIG)]
                + [jax.ShapeDtypeStruct((B, D), jnp.float32)
                   for _ in range(NSC)])
    mesh = plsc.VectorSubcoreMesh(core_axis_name="c", subcore_axis_name="s")
    fn = pl.kernel(
        _sc_gather_body,
        out_type=out_type,
        mesh=mesh,
        scratch_types=[
            pltpu.VMEM((NTASK, 128), jnp.int32),
            pltpu.VMEM((RING, 128, D), jnp.float32),
            pltpu.VMEM((2, 128), jnp.int32),
            pltpu.VMEM((2, 128, D), jnp.float32),
            pltpu.SemaphoreType.DMA,
            pltpu.SemaphoreType.DMA,
            pltpu.SemaphoreType.DMA,
        ],
    )
    return fn(*tabs, idx_all, *stabs, *sidxs)


def _tc_body(BB, *refs):
    name_ref = refs[0]
    scal = refs[1:1 + NSC]
    seqs = refs[1 + NSC:1 + NSC + 9]
    wxs = refs[1 + NSC + 9:1 + NSC + 18]
    whs = refs[1 + NSC + 18:1 + NSC + 27]
    bs = refs[1 + NSC + 27:1 + NSC + 36]
    W0, b0, W1, b1, W2, b2 = refs[1 + NSC + 36:1 + NSC + 42]
    out_ref = refs[1 + NSC + 42]
    (x_ref,) = refs[1 + NSC + 43:]

    f32 = jnp.float32
    x_ref[:, 0:D] = jnp.mean(name_ref[...], axis=0)
    for j in range(NSC):
        x_ref[:, (1 + j) * D:(2 + j) * D] = scal[j][...]

    def step(t, hs):
        new = []
        for f in range(9):
            h = hs[f]
            xt = (jnp.dot(seqs[f][t], wxs[f][...], preferred_element_type=f32)
                  + bs[f][...])
            hg = jnp.dot(h, whs[f][:, :2 * D], preferred_element_type=f32)
            z = jax.nn.sigmoid(xt[:, :D] + hg[:, :D])
            r = jax.nn.sigmoid(xt[:, D:2 * D] + hg[:, D:])
            hh = jnp.tanh(xt[:, 2 * D:]
                          + jnp.dot(r * h, whs[f][:, 2 * D:],
                                    preferred_element_type=f32))
            new.append(z * h + (1.0 - z) * hh)
        return tuple(new)

    hs = lax.fori_loop(0, L, step,
                       tuple(jnp.zeros((BB, D), f32) for _ in range(9)))
    for f in range(9):
        x_ref[:, (7 + f) * D:(8 + f) * D] = hs[f]

    x = x_ref[...]
    y = jax.nn.relu(jnp.dot(x, W0[...], preferred_element_type=f32) + b0[...])
    y = jax.nn.relu(jnp.dot(y, W1[...], preferred_element_type=f32) + b1[...])
    out_ref[...] = jnp.dot(y, W2[...], preferred_element_type=f32) + b2[...]


def _tc_forward(name_g, scal_g, seq_g, wxs, whs, bs, dense):
    BB = 128
    grid = (B // BB,)
    time_spec = pl.BlockSpec((L, BB, D), lambda i: (0, i, 0))
    row_spec = pl.BlockSpec((BB, D), lambda i: (i, 0))

    def full(shape):
        n = len(shape)
        return pl.BlockSpec(shape, lambda i, n=n: (0,) * n)

    in_specs = ([time_spec] + [row_spec] * NSC + [time_spec] * 9
                + [full((D, 3 * D))] * 9 + [full((D, 3 * D))] * 9
                + [full((3 * D,))] * 9
                + [full(d.shape) for d in dense])
    out_spec = pl.BlockSpec((BB, LAYER_SIZES[-1]), lambda i: (i, 0))

    return pl.pallas_call(
        functools.partial(_tc_body, BB),
        grid=grid,
        in_specs=in_specs,
        out_specs=out_spec,
        out_shape=jax.ShapeDtypeStruct((B, LAYER_SIZES[-1]), jnp.float32),
        scratch_shapes=[
            pltpu.VMEM((BB, 16 * D), jnp.float32),
        ],
    )(name_g, *scal_g, *seq_g, *wxs, *whs, *bs, *dense)


def kernel(pl_name_tokens, pl_collaborative_idx, pl_pid_idx,
           duration_ms_seed_idx, n_songs_idx, n_artists_idx, n_albums_idx,
           artist_name_seq, track_uri_seq, track_name_seq,
           duration_ms_songs_seq, album_name_seq, artist_pop_seq,
           artists_followers_seq, track_pop_seq, artist_genres_seq, params):
    seq_idx = [artist_name_seq, track_uri_seq, track_name_seq,
               duration_ms_songs_seq, album_name_seq, artist_pop_seq,
               artists_followers_seq, track_pop_seq, artist_genres_seq]
    scal_idx = [pl_collaborative_idx, pl_pid_idx, duration_ms_seed_idx,
                n_songs_idx, n_artists_idx, n_albums_idx]

    big_names = ['pl_name'] + SEQ_FEATS
    big_idx = [pl_name_tokens] + seq_idx
    idxs = [jnp.reshape(jnp.swapaxes(a, 0, 1).astype(jnp.int32),
                        (NW, KCH, 128)) for a in big_idx]
    idx_all = jnp.reshape(jnp.stack(idxs, axis=1), (NW, NTASK, 128))
    sidxs = [jnp.reshape(a.astype(jnp.int32), (8, 1, 128)) for a in scal_idx]
    tabs = [params['tab_' + n] for n in big_names]
    stabs = [params['tab_' + n] for n in SCALAR_FEATS]

    g = _sc_gather(tabs, idx_all, stabs, sidxs)
    big_g = [jnp.reshape(a, (L, B, D)) for a in g[:NBIG]]
    scal_g = list(g[NBIG:])

    wxs = [params[f + '_Wx'] for f in SEQ_FEATS]
    whs = [params[f + '_Wh'] for f in SEQ_FEATS]
    bs = [params[f + '_b'] for f in SEQ_FEATS]
    dense = [params['dense_W0'], params['dense_b0'],
             params['dense_W1'], params['dense_b1'],
             params['dense_W2'], params['dense_b2']]

    return _tc_forward(big_g[0], scal_g, big_g[1:], wxs, whs, bs, dense)


# speedup vs baseline: 1.0370x; 1.0370x over previous
import functools

import jax
import jax.numpy as jnp
from jax import lax
from jax.experimental import pallas as pl
from jax.experimental.pallas import tpu as pltpu
from jax.experimental.pallas import tpu_sc as plsc

B = 1024
L = 20
D = 128
LAYER_SIZES = [512, 256, 128]
SCALAR_FEATS = ['pl_collaborative', 'pl_pid', 'duration_ms_seed', 'n_songs',
                'n_artists', 'n_albums']
SEQ_FEATS = ['artist_name', 'track_uri', 'track_name', 'duration_ms_songs',
             'album_name', 'artist_pop', 'artists_followers', 'track_pop',
             'artist_genres']

NC = 2
NS = 16
NW = NC * NS
NBIG = 10
ROWS_BIG = L * B
KCH = ROWS_BIG // NW // 128
NSC = 6


RING = 6
DEEP = 3
NTASK = NBIG * KCH


def _sc_gather_body(*refs):
    tabs = refs[0:NBIG]
    idx_all = refs[NBIG]
    stabs = refs[NBIG + 1:NBIG + 1 + NSC]
    sidxs = refs[NBIG + 1 + NSC:NBIG + 1 + 2 * NSC]
    outs = refs[NBIG + 1 + 2 * NSC:2 * NBIG + 1 + 2 * NSC]
    souts = refs[2 * NBIG + 1 + 2 * NSC:2 * NBIG + 1 + 3 * NSC]
    idx_v, rows_v, idx_s, rows_s, gsem, ssem, s2sem = \
        refs[2 * NBIG + 1 + 3 * NSC:]

    w = lax.axis_index("s") * NC + lax.axis_index("c")

    pltpu.sync_copy(idx_all.at[w], idx_v)

    def gcopy(i):
        f = i // KCH
        return pltpu.make_async_copy(tabs[f].at[idx_v.at[i]],
                                     rows_v.at[i % RING], gsem)

    def scopy(i):
        f, k = divmod(i, KCH)
        return pltpu.make_async_copy(rows_v.at[i % RING],
                                     outs[f].at[w * KCH + k], ssem)

    for i in range(DEEP):
        gcopy(i).start()
    for i in range(NTASK):
        gcopy(i).wait()
        scopy(i).start()
        if i >= DEEP:
            scopy(i - DEEP).wait()
        if i + DEEP < NTASK:
            gcopy(i + DEEP).start()
    for i in range(NTASK - DEEP, NTASK):
        scopy(i).wait()

    @pl.when(w < 8)
    def _():
        def sg(f, b):
            pltpu.sync_copy(sidxs[f].at[w], idx_s.at[pl.ds(b, 1)])
            pltpu.make_async_copy(stabs[f].at[idx_s.at[b]],
                                  rows_s, s2sem).start()

        def sg_wait(f, b):
            pltpu.make_async_copy(stabs[f].at[idx_s.at[b]],
                                  rows_s, s2sem).wait()

        sg(0, 0)
        for f in range(NSC):
            b = f % 2
            sg_wait(f, b)
            pltpu.sync_copy(rows_s, souts[f].at[pl.ds(w * 128, 128)])
            if f + 1 < NSC:
                sg(f + 1, 1 - b)


def _sc_gather(tabs, idx_all, stabs, sidxs):
    out_type = ([jax.ShapeDtypeStruct((ROWS_BIG // 128, 128, D), jnp.float32)
                 for _ in range(NBIG)]
                + [jax.ShapeDtypeStruct((B, D), jnp.float32)
                   for _ in range(NSC)])
    mesh = plsc.VectorSubcoreMesh(core_axis_name="c", subcore_axis_name="s")
    fn = pl.kernel(
        _sc_gather_body,
        out_type=out_type,
        mesh=mesh,
        scratch_types=[
            pltpu.VMEM((NTASK, 128), jnp.int32),
            pltpu.VMEM((RING, 128, D), jnp.float32),
            pltpu.VMEM((2, 128), jnp.int32),
            pltpu.VMEM((128, D), jnp.float32),
            pltpu.SemaphoreType.DMA,
            pltpu.SemaphoreType.DMA,
            pltpu.SemaphoreType.DMA,
        ],
    )
    return fn(*tabs, idx_all, *stabs, *sidxs)


def _tc_body(BB, *refs):
    name_ref = refs[0]
    scal = refs[1:1 + NSC]
    seqs = refs[1 + NSC:1 + NSC + 9]
    wxs = refs[1 + NSC + 9:1 + NSC + 18]
    whs = refs[1 + NSC + 18:1 + NSC + 27]
    bs = refs[1 + NSC + 27:1 + NSC + 36]
    W0, b0, W1, b1, W2, b2 = refs[1 + NSC + 36:1 + NSC + 42]
    out_ref = refs[1 + NSC + 42]
    (x_ref,) = refs[1 + NSC + 43:]

    f32 = jnp.float32
    x_ref[:, 0:D] = jnp.mean(name_ref[...], axis=0)
    for j in range(NSC):
        x_ref[:, (1 + j) * D:(2 + j) * D] = scal[j][...]

    def step(t, hs):
        new = []
        for f in range(9):
            h = hs[f]
            xt = (jnp.dot(seqs[f][t], wxs[f][...], preferred_element_type=f32)
                  + bs[f][...])
            hg = jnp.dot(h, whs[f][:, :2 * D], preferred_element_type=f32)
            z = jax.nn.sigmoid(xt[:, :D] + hg[:, :D])
            r = jax.nn.sigmoid(xt[:, D:2 * D] + hg[:, D:])
            hh = jnp.tanh(xt[:, 2 * D:]
                          + jnp.dot(r * h, whs[f][:, 2 * D:],
                                    preferred_element_type=f32))
            new.append(z * h + (1.0 - z) * hh)
        return tuple(new)

    hs = lax.fori_loop(0, L, step,
                       tuple(jnp.zeros((BB, D), f32) for _ in range(9)))
    for f in range(9):
        x_ref[:, (7 + f) * D:(8 + f) * D] = hs[f]

    x = x_ref[...]
    y = jax.nn.relu(jnp.dot(x, W0[...], preferred_element_type=f32) + b0[...])
    y = jax.nn.relu(jnp.dot(y, W1[...], preferred_element_type=f32) + b1[...])
    out_ref[...] = jnp.dot(y, W2[...], preferred_element_type=f32) + b2[...]


def _tc_forward(name_g, scal_g, seq_g, wxs, whs, bs, dense):
    BB = 128
    grid = (B // BB,)
    time_spec = pl.BlockSpec((L, BB, D), lambda i: (0, i, 0))
    row_spec = pl.BlockSpec((BB, D), lambda i: (i, 0))

    def full(shape):
        n = len(shape)
        return pl.BlockSpec(shape, lambda i, n=n: (0,) * n)

    in_specs = ([time_spec] + [row_spec] * NSC + [time_spec] * 9
                + [full((D, 3 * D))] * 9 + [full((D, 3 * D))] * 9
                + [full((3 * D,))] * 9
                + [full(d.shape) for d in dense])
    out_spec = pl.BlockSpec((BB, LAYER_SIZES[-1]), lambda i: (i, 0))

    return pl.pallas_call(
        functools.partial(_tc_body, BB),
        grid=grid,
        in_specs=in_specs,
        out_specs=out_spec,
        out_shape=jax.ShapeDtypeStruct((B, LAYER_SIZES[-1]), jnp.float32),
        scratch_shapes=[
            pltpu.VMEM((BB, 16 * D), jnp.float32),
        ],
    )(name_g, *scal_g, *seq_g, *wxs, *whs, *bs, *dense)


def kernel(pl_name_tokens, pl_collaborative_idx, pl_pid_idx,
           duration_ms_seed_idx, n_songs_idx, n_artists_idx, n_albums_idx,
           artist_name_seq, track_uri_seq, track_name_seq,
           duration_ms_songs_seq, album_name_seq, artist_pop_seq,
           artists_followers_seq, track_pop_seq, artist_genres_seq, params):
    seq_idx = [artist_name_seq, track_uri_seq, track_name_seq,
               duration_ms_songs_seq, album_name_seq, artist_pop_seq,
               artists_followers_seq, track_pop_seq, artist_genres_seq]
    scal_idx = [pl_collaborative_idx, pl_pid_idx, duration_ms_seed_idx,
                n_songs_idx, n_artists_idx, n_albums_idx]

    big_names = ['pl_name'] + SEQ_FEATS
    big_idx = [pl_name_tokens] + seq_idx
    idxs = [jnp.reshape(jnp.swapaxes(a, 0, 1).astype(jnp.int32),
                        (NW, KCH, 128)) for a in big_idx]
    idx_all = jnp.reshape(jnp.stack(idxs, axis=1), (NW, NTASK, 128))
    sidxs = [jnp.reshape(a.astype(jnp.int32), (8, 1, 128)) for a in scal_idx]
    tabs = [params['tab_' + n] for n in big_names]
    stabs = [params['tab_' + n] for n in SCALAR_FEATS]

    g = _sc_gather(tabs, idx_all, stabs, sidxs)
    big_g = [jnp.reshape(a, (L, B, D)) for a in g[:NBIG]]
    scal_g = list(g[NBIG:])

    wxs = [params[f + '_Wx'] for f in SEQ_FEATS]
    whs = [params[f + '_Wh'] for f in SEQ_FEATS]
    bs = [params[f + '_b'] for f in SEQ_FEATS]
    dense = [params['dense_W0'], params['dense_b0'],
             params['dense_W1'], params['dense_b1'],
             params['dense_W2'], params['dense_b2']]

    return _tc_forward(big_g[0], scal_g, big_g[1:], wxs, whs, bs, dense)
